# baseline (speedup 1.0000x reference)
import functools

import numpy as np

import jax
import jax.numpy as jnp
from jax import lax
from jax.experimental import pallas as pl
from jax.experimental.pallas import tpu as pltpu
from jax.experimental.pallas import tpu_sc as plsc

B = 128
V = 100000
L = 16
G = 16
GR = 8
W = 49920
CW = 3840
NCH = 26
CV = CW // L
TAILO = 2 * W
TAILW = V - TAILO
TV = TAILW // L
INTMAX = np.int32(2**31 - 1)


def _np_exponential_key1(n):
    x0 = np.zeros(n, np.uint32)
    x1 = np.arange(n, dtype=np.uint32)
    ks = [np.uint32(0), np.uint32(1),
          np.uint32(0x1BD11BDA) ^ np.uint32(0) ^ np.uint32(1)]
    rot_a = (13, 15, 26, 6)
    rot_b = (17, 29, 16, 24)

    def rot(x, d):
        return ((x << np.uint32(d)) | (x >> np.uint32(32 - d))).astype(
            np.uint32)

    x0 = (x0 + ks[0]).astype(np.uint32)
    x1 = (x1 + ks[1]).astype(np.uint32)
    for i in range(5):
        for d in (rot_a if i % 2 == 0 else rot_b):
            x0 = (x0 + x1).astype(np.uint32)
            x1 = rot(x1, d) ^ x0
        x0 = (x0 + ks[(i + 1) % 3]).astype(np.uint32)
        x1 = (x1 + ks[(i + 2) % 3] + np.uint32(i + 1)).astype(np.uint32)
    bits = x0 ^ x1
    f = ((bits >> np.uint32(9)) | np.uint32(0x3F800000)).view(np.float32)
    return -np.log1p(-(f - np.float32(1.0)))


@functools.cache
def _inv_q3():
    try:
        q = jax.random.exponential(jax.random.key(1), (B, V),
                                   dtype=jnp.float32)
        return jax.block_until_ready((1.0 / q).reshape(G, GR, V))
    except Exception:
        q = _np_exponential_key1(B * V)
        return (np.float32(1.0) / q).reshape(G, GR, V)


_mesh = plsc.VectorSubcoreMesh(core_axis_name="c", subcore_axis_name="s")


@functools.partial(
    pl.kernel,
    out_type=(
        jax.ShapeDtypeStruct((G, GR, V), jnp.float32),
        jax.ShapeDtypeStruct((G, GR, L), jnp.int32),
    ),
    mesh=_mesh,
    compiler_params=pltpu.CompilerParams(needs_layout_passes=False),
    scratch_types=[
        pltpu.VMEM((GR, CW), jnp.float32),
        pltpu.VMEM((GR, CW), jnp.float32),
        pltpu.VMEM((GR, TAILW), jnp.float32),
        pltpu.VMEM((GR, TAILW), jnp.float32),
        pltpu.VMEM((GR, L), jnp.int32),
        pltpu.VMEM((L,), jnp.float32),
        pltpu.VMEM((L,), jnp.int32),
        pltpu.VMEM_SHARED((8, 2, 3, L), jnp.float32),
        pltpu.VMEM_SHARED((8, 2, 1, L), jnp.int32),
    ],
)
def _sampler_kernel(logits3, invq3, probs3, samp3,
                    a_v, q_v, at_v, qt_v, stage_v, xf_v, xi_v, shf, shi):
    c = lax.axis_index("c")
    s = lax.axis_index("s")
    gl = lax.shift_right_logical(s, 1)
    h = lax.bitwise_and(s, 1)
    g = c * 8 + gl
    base = h * 0
    lane = lax.iota(jnp.int32, L)
    ninf = jnp.full((L,), -jnp.inf, jnp.float32)

    def exf(i, vec):
        return jnp.broadcast_to(jnp.max(jnp.where(lane == i, vec, -jnp.inf)),
                                (L,))

    def exi(i, vec):
        return jnp.min(jnp.where(lane == i, vec, INTMAX))

    def p1_chunk(j, m8):
        off = base + j * CW
        pltpu.sync_copy(logits3.at[g, :, pl.ds(off, CW)], a_v)
        out = []
        for i in range(GR):
            def b(k, mm, i=i):
                ma, mb = mm
                o = k * (2 * L)
                ma = jnp.maximum(ma, a_v[i, pl.ds(o, L)])
                mb = jnp.maximum(mb, a_v[i, pl.ds(o + L, L)])
                return (ma, mb)
            out.append(lax.fori_loop(0, CV // 2, b, (m8[2 * i], m8[2 * i + 1]),
                                     unroll=4))
        return tuple(x for pair in out for x in pair)

    m8 = lax.fori_loop(0, NCH, p1_chunk, (ninf,) * (2 * GR))
    pltpu.sync_copy(logits3.at[g, :, pl.ds(TAILO, TAILW)], at_v)
    mrow = []
    for i in range(GR):
        def bt(k, mm, i=i):
            return jnp.maximum(mm, at_v[i, pl.ds(k * L, L)])
        mt = lax.fori_loop(0, TV, bt, jnp.maximum(m8[2 * i], m8[2 * i + 1]))
        mrow.append(mt)

    mx = ninf
    for i in range(GR):
        mx = jnp.where(lane == i, jnp.broadcast_to(jnp.max(mrow[i]), (L,)), mx)
    mboth = mx
    mb = [exf(i, mboth) for i in range(GR)]

    zsum = jnp.zeros((L,), jnp.float32)
    zidx = jnp.zeros((L,), jnp.int32)
    neg1 = jnp.full((L,), -1.0, jnp.float32)

    def merge(va, ia, vb, ib):
        u = (vb > va) | ((vb == va) & (ib < ia))
        return jnp.where(u, vb, va), jnp.where(u, ib, ia)

    def p2_chunk(j, carry):
        off = base + j * CW
        pltpu.sync_copy(logits3.at[g, :, pl.ds(off, CW)], a_v)
        pltpu.sync_copy(invq3.at[g, :, pl.ds(off, CW)], q_v)
        out = []
        for i in range(GR):
            def b(k, cr, i=i):
                sa, sb, ra, rb, ia, ib = cr
                o = k * (2 * L)
                x1 = a_v[i, pl.ds(o, L)]
                t1 = jnp.exp(x1 - mb[i])
                rv1 = t1 * q_v[i, pl.ds(o, L)]
                i1 = lane + (off + o)
                u1 = rv1 > ra
                x2 = a_v[i, pl.ds(o + L, L)]
                t2 = jnp.exp(x2 - mb[i])
                rv2 = t2 * q_v[i, pl.ds(o + L, L)]
                i2 = lane + (off + o + L)
                u2 = rv2 > rb
                return (sa + t1, sb + t2,
                        jnp.where(u1, rv1, ra), jnp.where(u2, rv2, rb),
                        jnp.where(u1, i1, ia), jnp.where(u2, i2, ib))
            s0, rm0, ri0 = carry[3 * i], carry[3 * i + 1], carry[3 * i + 2]
            sa, sb, ra, rb, ia, ib = lax.fori_loop(
                0, CV // 2, b, (zsum, zsum, neg1, neg1, zidx, zidx), unroll=4)
            cv, ci = merge(ra, ia, rb, ib)
            rm0, ri0 = merge(rm0, ri0, cv, ci)
            out.extend((s0 + sa + sb, rm0, ri0))
        return tuple(out)

    init2 = []
    for _ in range(GR):
        init2.extend((zsum, neg1, zidx))
    carry2 = lax.fori_loop(0, NCH, p2_chunk, tuple(init2))

    pltpu.sync_copy(invq3.at[g, :, pl.ds(TAILO, TAILW)], qt_v)
    zb = jnp.ones((L,), jnp.float32)
    rows = []
    for i in range(GR):
        def bt(k, cr, i=i):
            sa, ra, ia = cr
            o = k * L
            t1 = jnp.exp(at_v[i, pl.ds(o, L)] - mb[i])
            rv1 = t1 * qt_v[i, pl.ds(o, L)]
            i1 = lane + (TAILO + o)
            u1 = rv1 > ra
            return (sa + t1 * zb,
                    jnp.where(u1, rv1, ra), jnp.where(u1, i1, ia))
        s0, rm0, ri0 = carry2[3 * i], carry2[3 * i + 1], carry2[3 * i + 2]
        sa, ra, ia = lax.fori_loop(0, TV, bt, (zsum, neg1, zidx), unroll=2)
        rm0, ri0 = merge(rm0, ri0, ra, ia)
        rows.append((s0 + sa, rm0, ri0))

    psum, pval, pidx = zsum, neg1, zidx
    for i in range(GR):
        s0, rm0, ri0 = rows[i]
        psum = jnp.where(lane == i, jnp.broadcast_to(jnp.sum(s0), (L,)), psum)
        mi = jnp.max(rm0)
        cand = jnp.where(rm0 == mi, ri0, INTMAX)
        pval = jnp.where(lane == i, jnp.broadcast_to(mi, (L,)), pval)
        pidx = jnp.where(lane == i,
                         jnp.broadcast_to(jnp.min(cand), (L,)), pidx)

    stot = psum
    cvec = jnp.ones((L,), jnp.float32) / stot
    cb = [exf(i, cvec) for i in range(GR)]
    fval, fidx = pval, pidx

    @pl.when(h == 0)
    def _():
        for i in range(GR):
            stage_v[i, :] = jnp.broadcast_to(exi(i, fidx), (L,))
        pltpu.sync_copy(stage_v, samp3.at[g])

    def p3_chunk(j, _):
        off = base + j * CW
        pltpu.sync_copy(logits3.at[g, :, pl.ds(off, CW)], a_v)
        for i in range(GR):
            def b(k, _, i=i):
                o = k * L
                p = jnp.exp(a_v[i, pl.ds(o, L)] - mb[i]) * cb[i]
                a_v[i, pl.ds(o, L)] = p
                return 0
            lax.fori_loop(0, CV, b, 0, unroll=8)
        pltpu.sync_copy(a_v, probs3.at[g, :, pl.ds(off, CW)])
        return 0

    lax.fori_loop(0, NCH, p3_chunk, 0)
    for i in range(GR):
        def bt(k, _, i=i):
            o = k * L
            p = jnp.exp(at_v[i, pl.ds(o, L)] - mb[i]) * cb[i]
            at_v[i, pl.ds(o, L)] = p
            return 0
        lax.fori_loop(0, TV, bt, 0, unroll=2)
    pltpu.sync_copy(at_v, probs3.at[g, :, pl.ds(TAILO, TAILW)])


def kernel(logits):
    logits32 = logits.astype(jnp.float32)
    l3 = logits32.reshape(G, GR, V)
    probs3, samp3 = _sampler_kernel(l3, _inv_q3())
    probs = probs3.reshape(B, V)
    samp = samp3.reshape(B, L)[:, 0]
    return (logits32, probs, samp)


# speedup vs baseline: 1.1556x; 1.1556x over previous
import functools

import numpy as np

import jax
import jax.numpy as jnp
from jax import lax
from jax.experimental import pallas as pl
from jax.experimental.pallas import tpu as pltpu
from jax.experimental.pallas import tpu_sc as plsc

B = 128
V = 100000
L = 16
G = 16
GR = 8
CW = 3840
NCH = 26
CV = CW // L
TAILO = NCH * CW
TAILW = V - TAILO
TV = TAILW // L
INTMAX = np.int32(2**31 - 1)


def _np_exponential_key1(n):
    x0 = np.zeros(n, np.uint32)
    x1 = np.arange(n, dtype=np.uint32)
    ks = [np.uint32(0), np.uint32(1),
          np.uint32(0x1BD11BDA) ^ np.uint32(0) ^ np.uint32(1)]
    rot_a = (13, 15, 26, 6)
    rot_b = (17, 29, 16, 24)

    def rot(x, d):
        return ((x << np.uint32(d)) | (x >> np.uint32(32 - d))).astype(
            np.uint32)

    x0 = (x0 + ks[0]).astype(np.uint32)
    x1 = (x1 + ks[1]).astype(np.uint32)
    for i in range(5):
        for d in (rot_a if i % 2 == 0 else rot_b):
            x0 = (x0 + x1).astype(np.uint32)
            x1 = rot(x1, d) ^ x0
        x0 = (x0 + ks[(i + 1) % 3]).astype(np.uint32)
        x1 = (x1 + ks[(i + 2) % 3] + np.uint32(i + 1)).astype(np.uint32)
    bits = x0 ^ x1
    f = ((bits >> np.uint32(9)) | np.uint32(0x3F800000)).view(np.float32)
    return -np.log1p(-(f - np.float32(1.0)))


@functools.cache
def _inv_q3():
    try:
        q = jax.random.exponential(jax.random.key(1), (B, V),
                                   dtype=jnp.float32)
        return jax.block_until_ready((1.0 / q).reshape(G, GR, V))
    except Exception:
        q = _np_exponential_key1(B * V)
        return (np.float32(1.0) / q).reshape(G, GR, V)


_mesh = plsc.VectorSubcoreMesh(core_axis_name="c", subcore_axis_name="s")


@functools.partial(
    pl.kernel,
    out_type=(
        jax.ShapeDtypeStruct((G, GR, V), jnp.float32),
        jax.ShapeDtypeStruct((G, GR, L), jnp.int32),
    ),
    mesh=_mesh,
    compiler_params=pltpu.CompilerParams(needs_layout_passes=False),
    scratch_types=[
        pltpu.VMEM((GR, CW), jnp.float32),
        pltpu.VMEM((GR, CW), jnp.float32),
        pltpu.VMEM((GR, TAILW), jnp.float32),
        pltpu.VMEM((GR, TAILW), jnp.float32),
        pltpu.VMEM((GR, L), jnp.int32),
    ],
)
def _sampler_kernel(logits3, invq3, probs3, samp3,
                    a_v, q_v, at_v, qt_v, stage_v):
    c = lax.axis_index("c")
    s = lax.axis_index("s")
    gl = lax.shift_right_logical(s, 1)
    h = lax.bitwise_and(s, 1)
    g = c * 8 + gl
    lane = lax.iota(jnp.int32, L)
    ninf = jnp.full((L,), -jnp.inf, jnp.float32)
    zsum = jnp.zeros((L,), jnp.float32)
    zidx = jnp.zeros((L,), jnp.int32)
    neg1 = jnp.full((L,), -1.0, jnp.float32)

    def exf(i, vec):
        return jnp.broadcast_to(jnp.max(jnp.where(lane == i, vec, -jnp.inf)),
                                (L,))

    def p1_chunk(j, m8):
        off = j * CW
        pltpu.sync_copy(logits3.at[g, :, pl.ds(off, CW)], a_v)
        out = []
        for i in range(GR):
            def b(k, mm, i=i):
                ma, mb = mm
                o = k * (2 * L)
                ma = jnp.maximum(ma, a_v[i, pl.ds(o, L)])
                mb = jnp.maximum(mb, a_v[i, pl.ds(o + L, L)])
                return (ma, mb)
            out.append(lax.fori_loop(0, CV // 2, b, (m8[2 * i], m8[2 * i + 1]),
                                     unroll=4))
        return tuple(x for pair in out for x in pair)

    m8 = lax.fori_loop(0, NCH, p1_chunk, (ninf,) * (2 * GR))
    pltpu.sync_copy(logits3.at[g, :, pl.ds(TAILO, TAILW)], at_v)
    mb = []
    for i in range(GR):
        def bt(k, mm, i=i):
            return jnp.maximum(mm, at_v[i, pl.ds(k * L, L)])
        mt = lax.fori_loop(0, TV, bt, jnp.maximum(m8[2 * i], m8[2 * i + 1]))
        mb.append(jnp.broadcast_to(jnp.max(mt), (L,)))

    def merge(va, ia, vb, ib):
        u = (vb > va) | ((vb == va) & (ib < ia))
        return jnp.where(u, vb, va), jnp.where(u, ib, ia)

    @pl.when(h == 0)
    def _():
        def am_chunk(j, carry):
            off = j * CW
            pltpu.sync_copy(logits3.at[g, :, pl.ds(off, CW)], a_v)
            pltpu.sync_copy(invq3.at[g, :, pl.ds(off, CW)], q_v)
            out = []
            for i in range(GR):
                def b(k, cr, i=i):
                    ra, rb, ia, ib = cr
                    o = k * (2 * L)
                    rv1 = jnp.exp(a_v[i, pl.ds(o, L)] - mb[i]) \
                        * q_v[i, pl.ds(o, L)]
                    i1 = lane + (off + o)
                    u1 = rv1 > ra
                    rv2 = jnp.exp(a_v[i, pl.ds(o + L, L)] - mb[i]) \
                        * q_v[i, pl.ds(o + L, L)]
                    i2 = lane + (off + o + L)
                    u2 = rv2 > rb
                    return (jnp.where(u1, rv1, ra), jnp.where(u2, rv2, rb),
                            jnp.where(u1, i1, ia), jnp.where(u2, i2, ib))
                ra, rb, ia, ib = lax.fori_loop(
                    0, CV // 2, b, (neg1, neg1, zidx, zidx), unroll=4)
                cv, ci = merge(ra, ia, rb, ib)
                rm0, ri0 = merge(carry[2 * i], carry[2 * i + 1], cv, ci)
                out.extend((rm0, ri0))
            return tuple(out)

        init = []
        for _i in range(GR):
            init.extend((neg1, zidx))
        am = lax.fori_loop(0, NCH, am_chunk, tuple(init))

        pltpu.sync_copy(invq3.at[g, :, pl.ds(TAILO, TAILW)], qt_v)
        for i in range(GR):
            def bt(k, cr, i=i):
                ra, ia = cr
                o = k * L
                rv1 = jnp.exp(at_v[i, pl.ds(o, L)] - mb[i]) \
                    * qt_v[i, pl.ds(o, L)]
                i1 = lane + (TAILO + o)
                u1 = rv1 > ra
                return (jnp.where(u1, rv1, ra), jnp.where(u1, i1, ia))
            ra, ia = lax.fori_loop(0, TV, bt, (neg1, zidx), unroll=2)
            rm0, ri0 = merge(am[2 * i], am[2 * i + 1], ra, ia)
            mi = jnp.max(rm0)
            cand = jnp.where(rm0 == mi, ri0, INTMAX)
            stage_v[i, :] = jnp.broadcast_to(jnp.min(cand), (L,))
        pltpu.sync_copy(stage_v, samp3.at[g])

    @pl.when(h == 1)
    def _():
        def sum_chunk(j, s8):
            off = j * CW
            pltpu.sync_copy(logits3.at[g, :, pl.ds(off, CW)], a_v)
            out = []
            for i in range(GR):
                def b(k, ss, i=i):
                    sa, sb = ss
                    o = k * (2 * L)
                    sa = sa + jnp.exp(a_v[i, pl.ds(o, L)] - mb[i])
                    sb = sb + jnp.exp(a_v[i, pl.ds(o + L, L)] - mb[i])
                    return (sa, sb)
                out.append(lax.fori_loop(0, CV // 2, b,
                                         (s8[2 * i], s8[2 * i + 1]),
                                         unroll=4))
            return tuple(x for pair in out for x in pair)

        s8 = lax.fori_loop(0, NCH, sum_chunk, (zsum,) * (2 * GR))
        psum = zsum
        for i in range(GR):
            def bt(k, ss, i=i):
                return ss + jnp.exp(at_v[i, pl.ds(k * L, L)] - mb[i])
            st = lax.fori_loop(0, TV, bt, s8[2 * i] + s8[2 * i + 1])
            psum = jnp.where(lane == i,
                             jnp.broadcast_to(jnp.sum(st), (L,)), psum)
        cvec = jnp.ones((L,), jnp.float32) / psum
        cb = [exf(i, cvec) for i in range(GR)]

        def pr_chunk(j, _unused):
            off = j * CW
            pltpu.sync_copy(logits3.at[g, :, pl.ds(off, CW)], a_v)
            for i in range(GR):
                def b(k, _, i=i):
                    o = k * L
                    p = jnp.exp(a_v[i, pl.ds(o, L)] - mb[i]) * cb[i]
                    a_v[i, pl.ds(o, L)] = p
                    return 0
                lax.fori_loop(0, CV, b, 0, unroll=8)
            pltpu.sync_copy(a_v, probs3.at[g, :, pl.ds(off, CW)])
            return 0

        lax.fori_loop(0, NCH, pr_chunk, 0)
        for i in range(GR):
            def bt(k, _, i=i):
                o = k * L
                p = jnp.exp(at_v[i, pl.ds(o, L)] - mb[i]) * cb[i]
                at_v[i, pl.ds(o, L)] = p
                return 0
            lax.fori_loop(0, TV, bt, 0, unroll=2)
        pltpu.sync_copy(at_v, probs3.at[g, :, pl.ds(TAILO, TAILW)])


def kernel(logits):
    logits32 = logits.astype(jnp.float32)
    l3 = logits32.reshape(G, GR, V)
    probs3, samp3 = _sampler_kernel(l3, _inv_q3())
    probs = probs3.reshape(B, V)
    samp = samp3.reshape(B, L)[:, 0]
    return (logits32, probs, samp)
